# baseline (speedup 1.0000x reference)
import functools

import jax
import jax.numpy as jnp
from jax import lax
from jax.experimental import pallas as pl
from jax.experimental.pallas import tpu as pltpu
from jax.experimental.pallas import tpu_sc as plsc

_INFO = plsc.get_sparse_core_info()
_NC = _INFO.num_cores
_NS = _INFO.num_subcores
_NW = _NC * _NS

_U_PER_W, _U_BM = 3120, 240

_S = 4
_SB_I = 2500
_DEPTH = 4
_NBUF = 8


def _sc_user_body(u_ref, uo_ref, buf, in_sems, out_sems):
    wid = lax.axis_index("s") * _NC + lax.axis_index("c")

    def in_copy(off, bm, b):
        return pltpu.make_async_copy(
            u_ref.at[pl.ds(pl.multiple_of(off, 8), bm), :],
            buf.at[b, pl.ds(0, bm)],
            in_sems.at[b],
        )

    def out_copy(off, bm, b):
        return pltpu.make_async_copy(
            buf.at[b, pl.ds(0, bm)],
            uo_ref.at[pl.ds(pl.multiple_of(off, 8), bm), :],
            out_sems.at[b],
        )

    base = wid * _U_PER_W
    n = _U_PER_W // _U_BM
    in_copy(base, _U_BM, 0).start()
    for c in range(n):
        b = c % 2
        in_copy(base + c * _U_BM, _U_BM, b).wait()
        out_copy(base + c * _U_BM, _U_BM, b).start()
        if c + 1 < n:
            if c >= 1:
                out_copy(base + (c - 1) * _U_BM, _U_BM, 1 - b).wait()
            in_copy(base + (c + 1) * _U_BM, _U_BM, 1 - b).start()
    out_copy(base + (n - 1) * _U_BM, _U_BM, (n - 1) % 2).wait()

    @pl.when(wid == _NW - 1)
    def _tail():
        off, rows = _NW * _U_PER_W, 100000 - _NW * _U_PER_W
        in_copy(off, rows, 0).start()
        in_copy(off, rows, 0).wait()
        out_copy(off, rows, 0).start()
        out_copy(off, rows, 0).wait()


def _tc_item_body(i_ref, io_ref, buf, in_sems, out_sems):
    n, d = i_ref.shape
    ws = i_ref.reshape(_S, n // _S, d)
    wd = io_ref.reshape(_S, n // _S, d)
    n_chunks = (n // _S) // _SB_I

    def in_copy(c):
        b = c % _NBUF
        return pltpu.make_async_copy(
            ws.at[:, pl.ds(c * _SB_I, _SB_I), :], buf.at[b], in_sems.at[b]
        )

    def out_copy(c):
        b = c % _NBUF
        return pltpu.make_async_copy(
            buf.at[b], wd.at[:, pl.ds(c * _SB_I, _SB_I), :], out_sems.at[b]
        )

    for c in range(min(_DEPTH, n_chunks)):
        in_copy(c).start()
    for c in range(n_chunks):
        in_copy(c).wait()
        out_copy(c).start()
        nxt = c + _DEPTH
        if nxt < n_chunks:
            if nxt >= _NBUF:
                out_copy(nxt - _NBUF).wait()
            in_copy(nxt).start()
    for c in range(max(0, n_chunks - _NBUF), n_chunks):
        out_copy(c).wait()


def kernel(user_emb, item_emb):
    mesh = plsc.VectorSubcoreMesh(core_axis_name="c", subcore_axis_name="s")
    sc_user = functools.partial(
        pl.kernel,
        out_type=jax.ShapeDtypeStruct(user_emb.shape, user_emb.dtype),
        mesh=mesh,
        scratch_types=[
            pltpu.VMEM((2, _U_BM, 64), jnp.float32),
            pltpu.SemaphoreType.DMA((2,)),
            pltpu.SemaphoreType.DMA((2,)),
        ],
    )(_sc_user_body)
    u = sc_user(user_emb)

    i = pl.pallas_call(
        _tc_item_body,
        in_specs=[pl.BlockSpec(memory_space=pl.ANY)],
        out_specs=pl.BlockSpec(memory_space=pl.ANY),
        out_shape=jax.ShapeDtypeStruct(item_emb.shape, item_emb.dtype),
        scratch_shapes=[
            pltpu.VMEM((_NBUF, _S, _SB_I, 64), jnp.float32),
            pltpu.SemaphoreType.DMA((_NBUF,)),
            pltpu.SemaphoreType.DMA((_NBUF,)),
        ],
    )(item_emb)
    return (u, i)
